# baseline (speedup 1.0000x reference)
import jax
import jax.numpy as jnp
from jax import lax
from jax.experimental import pallas as pl
from jax.experimental.pallas import tpu as pltpu
from jax.experimental.pallas import tpu_sc as plsc

N = 10000
NP = 10240
FIN = 240
H = 512
OUT = 1200
E = 320000
EP = 327680
NC = 2
NS = 16
LN = 16
STRIPE = NP // NS
EPT = EP // NS
NB = EPT // 128
EPT_DEG = EP // (NC * NS)
NB_DEG = EPT_DEG // 128
PEDGE = 4096
PPARTS = EPT // PEDGE
BW = 128
PB = PEDGE // BW


def _sc_mesh():
    return plsc.VectorSubcoreMesh(
        core_axis_name="c", subcore_axis_name="s", num_cores=NC, num_subcores=NS
    )


def _deg_body(dst_hbm, out_hbm, acc, dstb, obuf, zbuf):
    c = lax.axis_index("c")
    s = lax.axis_index("s")
    z16 = jnp.zeros((LN,), jnp.float32)
    onehot = jnp.where(lax.iota(jnp.int32, LN) == 0, 1.0, 0.0).astype(jnp.float32)

    def initrow(r, carry):
        zbuf[r, pl.ds(0, LN)] = z16
        obuf[r, pl.ds(0, LN)] = onehot
        return carry

    lax.fori_loop(0, 128, initrow, 0)
    for z in range(STRIPE // 128):
        pltpu.sync_copy(zbuf, acc.at[pl.ds(s * STRIPE + z * 128, 128)])
    plsc.subcore_barrier()

    wid = c * NS + s

    def ebody(b, carry):
        base = wid * EPT_DEG + b * 128
        pltpu.sync_copy(dst_hbm.at[pl.ds(base, 128)], dstb.at[0])
        pltpu.sync_copy(obuf, acc.at[dstb.at[0]], add=True)
        return carry

    lax.fori_loop(0, NB_DEG, ebody, 0)
    plsc.subcore_barrier()
    pltpu.sync_copy(
        acc.at[pl.ds(s * STRIPE, STRIPE)],
        out_hbm.at[pl.ds(c * NP + s * STRIPE, STRIPE)],
    )


_deg_kernel = pl.kernel(
    _deg_body,
    out_type=jax.ShapeDtypeStruct((NC * NP, LN), jnp.float32),
    mesh=_sc_mesh(),
    scratch_types=[
        pltpu.VMEM_SHARED((NP, LN), jnp.float32),
        pltpu.VMEM((1, 128), jnp.int32),
        pltpu.VMEM((128, LN), jnp.float32),
        pltpu.VMEM((128, LN), jnp.float32),
    ],
)


def _make_prop(n_chunks):
    cps = n_chunks // NC

    def body(src_hbm, dst_hbm, table_hbm, out_hbm, acc, srcb, dstb,
             r0, r1, sg0, sg1, ss0, ss1):
        rows = (r0, r1)
        semg = (sg0, sg1)
        sems = (ss0, ss1)
        c = lax.axis_index("c")
        s = lax.axis_index("s")
        z16 = jnp.zeros((LN,), jnp.float32)

        def zrow(r, carry):
            for k in range(BW // LN):
                r0[r, pl.ds(k * LN, LN)] = z16
            return carry

        for cc in range(cps):
            roff = (cc * NC + c) * NP
            lax.fori_loop(0, BW, zrow, 0)
            for z in range(STRIPE // BW):
                pltpu.sync_copy(r0, acc.at[pl.ds(s * STRIPE + z * BW, BW)])
            plsc.subcore_barrier()

            def part(p, carry):
                base = s * EPT + p * PEDGE
                pltpu.sync_copy(src_hbm.at[pl.ds(base, PEDGE)], srcb)
                pltpu.sync_copy(dst_hbm.at[pl.ds(base, PEDGE)], dstb)

                def arow(r, c2):
                    sl = pl.ds(r * LN, LN)
                    srcb[sl] = srcb[sl] + roff
                    return c2

                lax.fori_loop(0, PEDGE // LN, arow, 0)

                gd = [None, None]
                sd = [None, None]
                gd[0] = pltpu.async_copy(
                    table_hbm.at[srcb.at[pl.ds(0, BW)]], rows[0], semg[0]
                )
                for b in range(PB):
                    j = b % 2
                    gd[j].wait()
                    sd[j] = pltpu.async_copy(
                        rows[j], acc.at[dstb.at[pl.ds(b * BW, BW)]],
                        sems[j], add=True,
                    )
                    if b + 1 < PB:
                        if b >= 1:
                            sd[1 - j].wait()
                        gd[1 - j] = pltpu.async_copy(
                            table_hbm.at[srcb.at[pl.ds((b + 1) * BW, BW)]],
                            rows[1 - j], semg[1 - j],
                        )
                sd[(PB - 2) % 2].wait()
                sd[(PB - 1) % 2].wait()
                return carry

            lax.fori_loop(0, PPARTS, part, 0)
            plsc.subcore_barrier()
            pltpu.sync_copy(
                acc.at[pl.ds(s * STRIPE, STRIPE)],
                out_hbm.at[pl.ds(roff + s * STRIPE, STRIPE)],
            )

    return pl.kernel(
        body,
        out_type=jax.ShapeDtypeStruct((n_chunks * NP, 128), jnp.float32),
        mesh=_sc_mesh(),
        scratch_types=[
            pltpu.VMEM_SHARED((NP, 128), jnp.float32),
            pltpu.VMEM((PEDGE,), jnp.int32),
            pltpu.VMEM((PEDGE,), jnp.int32),
            pltpu.VMEM((BW, 128), jnp.float32),
            pltpu.VMEM((BW, 128), jnp.float32),
            pltpu.SemaphoreType.DMA,
            pltpu.SemaphoreType.DMA,
            pltpu.SemaphoreType.DMA,
            pltpu.SemaphoreType.DMA,
        ],
    )


_prop2 = _make_prop(2)
_prop4 = _make_prop(4)


BM1 = 1280
BM2 = 1024
BM3 = 1000


def _scale_body(deg_ref, x_ref, y1_ref, dinv_ref):
    deg = deg_ref[0, :, 0:1] + deg_ref[1, :, 0:1]
    dinv = 1.0 / jnp.sqrt(1.0 + deg)
    y = x_ref[...] * dinv
    y1_ref[0] = y[:, :128]
    y1_ref[1] = jnp.pad(y[:, 128:], ((0, 0), (0, 256 - FIN)))
    dinv_ref[...] = jnp.broadcast_to(dinv, (BM1, 128))


_scale = pl.pallas_call(
    _scale_body,
    grid=(NP // BM1,),
    in_specs=[
        pl.BlockSpec((NC, BM1, LN), lambda i: (0, i, 0)),
        pl.BlockSpec((BM1, FIN), lambda i: (i, 0)),
    ],
    out_specs=[
        pl.BlockSpec((2, BM1, 128), lambda i: (0, i, 0)),
        pl.BlockSpec((BM1, 128), lambda i: (i, 0)),
    ],
    out_shape=[
        jax.ShapeDtypeStruct((2, NP, 128), jnp.float32),
        jax.ShapeDtypeStruct((NP, 128), jnp.float32),
    ],
)


def _mm1_body(z1_ref, y1_ref, dinv_ref, w_ref, b_ref, y2_ref):
    dinv = dinv_ref[:, 0:1]
    a = jnp.concatenate(
        [z1_ref[0] + y1_ref[0], (z1_ref[1] + y1_ref[1])[:, : FIN - 128]], axis=1
    ) * dinv
    h = lax.dot_general(
        a, w_ref[...], (((1,), (1,)), ((), ())), preferred_element_type=jnp.float32
    )
    h = jnp.maximum(h + b_ref[...], 0.0) * dinv
    for k in range(4):
        y2_ref[k] = h[:, k * 128 : (k + 1) * 128]


_mm1 = pl.pallas_call(
    _mm1_body,
    grid=(NP // BM2,),
    in_specs=[
        pl.BlockSpec((2, BM2, 128), lambda i: (0, i, 0)),
        pl.BlockSpec((2, BM2, 128), lambda i: (0, i, 0)),
        pl.BlockSpec((BM2, 128), lambda i: (i, 0)),
        pl.BlockSpec((H, FIN), lambda i: (0, 0)),
        pl.BlockSpec((1, H), lambda i: (0, 0)),
    ],
    out_specs=pl.BlockSpec((4, BM2, 128), lambda i: (0, i, 0)),
    out_shape=jax.ShapeDtypeStruct((4, NP, 128), jnp.float32),
)


def _wc_body(w3_ref, w2_ref, b2_ref, b3_ref, wc_ref, bc_ref):
    wc_ref[...] = lax.dot_general(
        w3_ref[...], w2_ref[...], (((1,), (0,)), ((), ())),
        preferred_element_type=jnp.float32,
    )
    bc_ref[...] = (
        lax.dot_general(
            b2_ref[...], w3_ref[...], (((1,), (1,)), ((), ())),
            preferred_element_type=jnp.float32,
        )
        + b3_ref[...]
    )


_wc = pl.pallas_call(
    _wc_body,
    in_specs=[
        pl.BlockSpec((OUT, OUT), lambda: (0, 0)),
        pl.BlockSpec((OUT, H), lambda: (0, 0)),
        pl.BlockSpec((1, OUT), lambda: (0, 0)),
        pl.BlockSpec((1, OUT), lambda: (0, 0)),
    ],
    out_specs=[
        pl.BlockSpec((OUT, H), lambda: (0, 0)),
        pl.BlockSpec((1, OUT), lambda: (0, 0)),
    ],
    out_shape=[
        jax.ShapeDtypeStruct((OUT, H), jnp.float32),
        jax.ShapeDtypeStruct((1, OUT), jnp.float32),
    ],
)


def _mm2_body(z2_ref, y2_ref, dinv_ref, wc_ref, bc_ref, o_ref):
    dinv = dinv_ref[:, 0:1]
    a = jnp.concatenate(
        [z2_ref[k] + y2_ref[k] for k in range(4)], axis=1
    ) * dinv
    o_ref[...] = (
        lax.dot_general(
            a, wc_ref[...], (((1,), (1,)), ((), ())),
            preferred_element_type=jnp.float32,
        )
        + bc_ref[...]
    )


_mm2 = pl.pallas_call(
    _mm2_body,
    grid=(N // BM3,),
    in_specs=[
        pl.BlockSpec((4, BM3, 128), lambda i: (0, i, 0)),
        pl.BlockSpec((4, BM3, 128), lambda i: (0, i, 0)),
        pl.BlockSpec((BM3, 128), lambda i: (i, 0)),
        pl.BlockSpec((OUT, H), lambda i: (0, 0)),
        pl.BlockSpec((1, OUT), lambda i: (0, 0)),
    ],
    out_specs=pl.BlockSpec((BM3, OUT), lambda i: (i, 0)),
    out_shape=jax.ShapeDtypeStruct((N, OUT), jnp.float32),
)


def kernel(x, edge_index, W1, b1, W2, b2, W3, b3):
    pad = jnp.full((EP - E,), N, jnp.int32)
    srcp = jnp.concatenate([edge_index[0].astype(jnp.int32), pad])
    dstp = jnp.concatenate([edge_index[1].astype(jnp.int32), pad])

    degp = _deg_kernel(dstp).reshape(NC, NP, LN)
    y1, dinvb = _scale(degp, x)
    z1 = _prop2(srcp, dstp, y1.reshape(2 * NP, 128)).reshape(2, NP, 128)
    y2 = _mm1(z1, y1, dinvb, W1, b1.reshape(1, H))
    z2 = _prop4(srcp, dstp, y2.reshape(4 * NP, 128)).reshape(4, NP, 128)
    wc, bc = _wc(W3, W2, b2.reshape(1, OUT), b3.reshape(1, OUT))
    return _mm2(z2, y2, dinvb, wc, bc)


# speedup vs baseline: 1.0386x; 1.0386x over previous
import jax
import jax.numpy as jnp
from jax import lax
from jax.experimental import pallas as pl
from jax.experimental.pallas import tpu as pltpu
from jax.experimental.pallas import tpu_sc as plsc

N = 10000
NP = 10240
FIN = 240
H = 512
OUT = 1200
E = 320000
EP = 327680
NC = 2
NS = 16
LN = 16
STRIPE = NP // NS
EPT = EP // NS
NB = EPT // 128
EPT_DEG = EP // (NC * NS)
NB_DEG = EPT_DEG // 128
PEDGE = 4096
PPARTS = EPT // PEDGE
BW = 64
PB = PEDGE // BW


def _sc_mesh():
    return plsc.VectorSubcoreMesh(
        core_axis_name="c", subcore_axis_name="s", num_cores=NC, num_subcores=NS
    )


def _deg_body(dst_hbm, out_hbm, acc, dstb, obuf, zbuf):
    c = lax.axis_index("c")
    s = lax.axis_index("s")
    z16 = jnp.zeros((LN,), jnp.float32)
    onehot = jnp.where(lax.iota(jnp.int32, LN) == 0, 1.0, 0.0).astype(jnp.float32)

    def initrow(r, carry):
        zbuf[r, pl.ds(0, LN)] = z16
        obuf[r, pl.ds(0, LN)] = onehot
        return carry

    lax.fori_loop(0, 128, initrow, 0)
    for z in range(STRIPE // 128):
        pltpu.sync_copy(zbuf, acc.at[pl.ds(s * STRIPE + z * 128, 128)])
    plsc.subcore_barrier()

    wid = c * NS + s

    def ebody(b, carry):
        base = wid * EPT_DEG + b * 128
        pltpu.sync_copy(dst_hbm.at[pl.ds(base, 128)], dstb.at[0])
        pltpu.sync_copy(obuf, acc.at[dstb.at[0]], add=True)
        return carry

    lax.fori_loop(0, NB_DEG, ebody, 0)
    plsc.subcore_barrier()
    pltpu.sync_copy(
        acc.at[pl.ds(s * STRIPE, STRIPE)],
        out_hbm.at[pl.ds(c * NP + s * STRIPE, STRIPE)],
    )


_deg_kernel = pl.kernel(
    _deg_body,
    out_type=jax.ShapeDtypeStruct((NC * NP, LN), jnp.float32),
    mesh=_sc_mesh(),
    scratch_types=[
        pltpu.VMEM_SHARED((NP, LN), jnp.float32),
        pltpu.VMEM((1, 128), jnp.int32),
        pltpu.VMEM((128, LN), jnp.float32),
        pltpu.VMEM((128, LN), jnp.float32),
    ],
)


def _make_prop(n_chunks):
    cps = n_chunks // NC

    def body(src_hbm, dst_hbm, table_hbm, out_hbm, acc, srcb, dstb,
             r0, r1, r2, r3, sg0, sg1, sg2, sg3, ss0, ss1, ss2, ss3):
        rows = (r0, r1, r2, r3)
        semg = (sg0, sg1, sg2, sg3)
        sems = (ss0, ss1, ss2, ss3)
        c = lax.axis_index("c")
        s = lax.axis_index("s")
        z16 = jnp.zeros((LN,), jnp.float32)

        def zrow(r, carry):
            for k in range(BW // LN):
                r0[r, pl.ds(k * LN, LN)] = z16
            return carry

        for cc in range(cps):
            roff = (cc * NC + c) * NP
            lax.fori_loop(0, BW, zrow, 0)
            for z in range(STRIPE // BW):
                pltpu.sync_copy(r0, acc.at[pl.ds(s * STRIPE + z * BW, BW)])
            plsc.subcore_barrier()

            def part(p, carry):
                base = s * EPT + p * PEDGE
                pltpu.sync_copy(src_hbm.at[pl.ds(base, PEDGE)], srcb)
                pltpu.sync_copy(dst_hbm.at[pl.ds(base, PEDGE)], dstb)

                def arow(r, c2):
                    sl = pl.ds(r * LN, LN)
                    srcb[sl] = srcb[sl] + roff
                    return c2

                lax.fori_loop(0, PEDGE // LN, arow, 0)

                gd = [None, None, None, None]
                sd = [None, None, None, None]
                for b in range(2):
                    gd[b] = pltpu.async_copy(
                        table_hbm.at[srcb.at[pl.ds(b * BW, BW)]],
                        rows[b], semg[b],
                    )
                for b in range(PB):
                    j = b % 4
                    gd[j].wait()
                    sd[j] = pltpu.async_copy(
                        rows[j], acc.at[dstb.at[pl.ds(b * BW, BW)]],
                        sems[j], add=True,
                    )
                    jp = (b + 2) % 4
                    if b >= 2:
                        sd[jp].wait()
                    if b + 2 < PB:
                        gd[jp] = pltpu.async_copy(
                            table_hbm.at[srcb.at[pl.ds((b + 2) * BW, BW)]],
                            rows[jp], semg[jp],
                        )
                sd[(PB - 2) % 4].wait()
                sd[(PB - 1) % 4].wait()
                return carry

            lax.fori_loop(0, PPARTS, part, 0)
            plsc.subcore_barrier()
            pltpu.sync_copy(
                acc.at[pl.ds(s * STRIPE, STRIPE)],
                out_hbm.at[pl.ds(roff + s * STRIPE, STRIPE)],
            )

    return pl.kernel(
        body,
        out_type=jax.ShapeDtypeStruct((n_chunks * NP, 128), jnp.float32),
        mesh=_sc_mesh(),
        scratch_types=[
            pltpu.VMEM_SHARED((NP, 128), jnp.float32),
            pltpu.VMEM((PEDGE,), jnp.int32),
            pltpu.VMEM((PEDGE,), jnp.int32),
            pltpu.VMEM((BW, 128), jnp.float32),
            pltpu.VMEM((BW, 128), jnp.float32),
            pltpu.VMEM((BW, 128), jnp.float32),
            pltpu.VMEM((BW, 128), jnp.float32),
            pltpu.SemaphoreType.DMA,
            pltpu.SemaphoreType.DMA,
            pltpu.SemaphoreType.DMA,
            pltpu.SemaphoreType.DMA,
            pltpu.SemaphoreType.DMA,
            pltpu.SemaphoreType.DMA,
            pltpu.SemaphoreType.DMA,
            pltpu.SemaphoreType.DMA,
        ],
    )


_prop2 = _make_prop(2)
_prop4 = _make_prop(4)


BM1 = 1280
BM2 = 1024
BM3 = 1000


def _scale_body(deg_ref, x_ref, y1_ref, dinv_ref):
    deg = deg_ref[0, :, 0:1] + deg_ref[1, :, 0:1]
    dinv = 1.0 / jnp.sqrt(1.0 + deg)
    y = x_ref[...] * dinv
    y1_ref[0] = y[:, :128]
    y1_ref[1] = jnp.pad(y[:, 128:], ((0, 0), (0, 256 - FIN)))
    dinv_ref[...] = jnp.broadcast_to(dinv, (BM1, 128))


_scale = pl.pallas_call(
    _scale_body,
    grid=(NP // BM1,),
    in_specs=[
        pl.BlockSpec((NC, BM1, LN), lambda i: (0, i, 0)),
        pl.BlockSpec((BM1, FIN), lambda i: (i, 0)),
    ],
    out_specs=[
        pl.BlockSpec((2, BM1, 128), lambda i: (0, i, 0)),
        pl.BlockSpec((BM1, 128), lambda i: (i, 0)),
    ],
    out_shape=[
        jax.ShapeDtypeStruct((2, NP, 128), jnp.float32),
        jax.ShapeDtypeStruct((NP, 128), jnp.float32),
    ],
)


def _mm1_body(z1_ref, y1_ref, dinv_ref, w_ref, b_ref, y2_ref):
    dinv = dinv_ref[:, 0:1]
    a = jnp.concatenate(
        [z1_ref[0] + y1_ref[0], (z1_ref[1] + y1_ref[1])[:, : FIN - 128]], axis=1
    ) * dinv
    h = lax.dot_general(
        a, w_ref[...], (((1,), (1,)), ((), ())), preferred_element_type=jnp.float32
    )
    h = jnp.maximum(h + b_ref[...], 0.0) * dinv
    for k in range(4):
        y2_ref[k] = h[:, k * 128 : (k + 1) * 128]


_mm1 = pl.pallas_call(
    _mm1_body,
    grid=(NP // BM2,),
    in_specs=[
        pl.BlockSpec((2, BM2, 128), lambda i: (0, i, 0)),
        pl.BlockSpec((2, BM2, 128), lambda i: (0, i, 0)),
        pl.BlockSpec((BM2, 128), lambda i: (i, 0)),
        pl.BlockSpec((H, FIN), lambda i: (0, 0)),
        pl.BlockSpec((1, H), lambda i: (0, 0)),
    ],
    out_specs=pl.BlockSpec((4, BM2, 128), lambda i: (0, i, 0)),
    out_shape=jax.ShapeDtypeStruct((4, NP, 128), jnp.float32),
)


def _wc_body(w3_ref, w2_ref, b2_ref, b3_ref, wc_ref, bc_ref):
    wc_ref[...] = lax.dot_general(
        w3_ref[...], w2_ref[...], (((1,), (0,)), ((), ())),
        preferred_element_type=jnp.float32,
    )
    bc_ref[...] = (
        lax.dot_general(
            b2_ref[...], w3_ref[...], (((1,), (1,)), ((), ())),
            preferred_element_type=jnp.float32,
        )
        + b3_ref[...]
    )


_wc = pl.pallas_call(
    _wc_body,
    in_specs=[
        pl.BlockSpec((OUT, OUT), lambda: (0, 0)),
        pl.BlockSpec((OUT, H), lambda: (0, 0)),
        pl.BlockSpec((1, OUT), lambda: (0, 0)),
        pl.BlockSpec((1, OUT), lambda: (0, 0)),
    ],
    out_specs=[
        pl.BlockSpec((OUT, H), lambda: (0, 0)),
        pl.BlockSpec((1, OUT), lambda: (0, 0)),
    ],
    out_shape=[
        jax.ShapeDtypeStruct((OUT, H), jnp.float32),
        jax.ShapeDtypeStruct((1, OUT), jnp.float32),
    ],
)


def _mm2_body(z2_ref, y2_ref, dinv_ref, wc_ref, bc_ref, o_ref):
    dinv = dinv_ref[:, 0:1]
    a = jnp.concatenate(
        [z2_ref[k] + y2_ref[k] for k in range(4)], axis=1
    ) * dinv
    o_ref[...] = (
        lax.dot_general(
            a, wc_ref[...], (((1,), (1,)), ((), ())),
            preferred_element_type=jnp.float32,
        )
        + bc_ref[...]
    )


_mm2 = pl.pallas_call(
    _mm2_body,
    grid=(N // BM3,),
    in_specs=[
        pl.BlockSpec((4, BM3, 128), lambda i: (0, i, 0)),
        pl.BlockSpec((4, BM3, 128), lambda i: (0, i, 0)),
        pl.BlockSpec((BM3, 128), lambda i: (i, 0)),
        pl.BlockSpec((OUT, H), lambda i: (0, 0)),
        pl.BlockSpec((1, OUT), lambda i: (0, 0)),
    ],
    out_specs=pl.BlockSpec((BM3, OUT), lambda i: (i, 0)),
    out_shape=jax.ShapeDtypeStruct((N, OUT), jnp.float32),
)


def kernel(x, edge_index, W1, b1, W2, b2, W3, b3):
    pad = jnp.full((EP - E,), N, jnp.int32)
    srcp = jnp.concatenate([edge_index[0].astype(jnp.int32), pad])
    dstp = jnp.concatenate([edge_index[1].astype(jnp.int32), pad])

    degp = _deg_kernel(dstp).reshape(NC, NP, LN)
    y1, dinvb = _scale(degp, x)
    z1 = _prop2(srcp, dstp, y1.reshape(2 * NP, 128)).reshape(2, NP, 128)
    y2 = _mm1(z1, y1, dinvb, W1, b1.reshape(1, H))
    z2 = _prop4(srcp, dstp, y2.reshape(4 * NP, 128)).reshape(4, NP, 128)
    wc, bc = _wc(W3, W2, b2.reshape(1, OUT), b3.reshape(1, OUT))
    return _mm2(z2, y2, dinvb, wc, bc)
